# baseline (speedup 1.0000x reference)
import functools

import jax
import jax.numpy as jnp
from jax import lax
from jax.experimental import pallas as pl
from jax.experimental.pallas import tpu as pltpu
from jax.experimental.pallas import tpu_sc as plsc

VOCAB = 4100
DIM = 2048
B = 4 * 8192
NW = 32
BPW = B // NW
CH = 16
NCH = BPW // CH
NBUF = 3
NITER = NCH // NBUF
REMC = NCH - NITER * NBUF


def _gather_body(idx_hbm, table_hbm, out_hbm, idx_v, *scratch):
    rows = scratch[:NBUF]
    gsem = scratch[NBUF:2 * NBUF]
    wsem = scratch[2 * NBUF:]
    wid = lax.axis_index("s") * 2 + lax.axis_index("c")
    base = wid * BPW
    pltpu.sync_copy(idx_hbm.at[wid], idx_v)

    def fire_gather(j, b):
        return pltpu.async_copy(table_hbm.at[idx_v.at[j]], rows[b], gsem[b])

    def fire_wb(j, b):
        return pltpu.async_copy(
            rows[b], out_hbm.at[pl.ds(base + j * CH, CH)], wsem[b])

    def wait_gather(j, b):
        pltpu.make_async_copy(table_hbm.at[idx_v.at[j]], rows[b], gsem[b]).wait()

    def wait_wb(j, b):
        pltpu.make_async_copy(
            rows[b], out_hbm.at[pl.ds(base + j * CH, CH)], wsem[b]).wait()

    def body(i, _):
        j0 = i * NBUF
        for b in range(NBUF):
            @pl.when(i > 0)
            def _(b=b):
                wait_wb(j0 - NBUF + b, b)
            fire_gather(j0 + b, b)
        for b in range(NBUF):
            wait_gather(j0 + b, b)
            fire_wb(j0 + b, b)
        return 0

    lax.fori_loop(0, NITER, body, 0)
    last = {b: (NITER - 1) * NBUF + b for b in range(NBUF)}
    for b in range(REMC):
        j = NITER * NBUF + b
        wait_wb(last[b], b)
        fire_gather(j, b)
        wait_gather(j, b)
        fire_wb(j, b)
        last[b] = j
    for b in range(NBUF):
        wait_wb(last[b], b)


@jax.jit
def _run(idx, table):
    mesh = plsc.VectorSubcoreMesh(core_axis_name="c", subcore_axis_name="s")
    f = functools.partial(
        pl.kernel,
        mesh=mesh,
        out_type=jax.ShapeDtypeStruct((B, DIM), jnp.float32),
        scratch_types=(
            [pltpu.VMEM((NCH, CH), jnp.int32)]
            + [pltpu.VMEM((CH, DIM), jnp.float32) for _ in range(NBUF)]
            + [pltpu.SemaphoreType.DMA for _ in range(2 * NBUF)]
        ),
    )(_gather_body)
    return f(idx, table)


def kernel(DNA, table):
    idx = DNA.reshape(NW, NCH, CH)
    out = _run(idx, table)
    return out.reshape(4, 8192, DIM)


# speedup vs baseline: 1.0052x; 1.0052x over previous
import functools

import jax
import jax.numpy as jnp
from jax import lax
from jax.experimental import pallas as pl
from jax.experimental.pallas import tpu as pltpu
from jax.experimental.pallas import tpu_sc as plsc

VOCAB = 4100
DIM = 2048
B = 4 * 8192
NW = 32
BPW = B // NW
CH = 8
NCH = BPW // CH
NBUF = 6
NITER = NCH // NBUF
REMC = NCH - NITER * NBUF


def _gather_body(idx_hbm, table_hbm, out_hbm, idx_v, *scratch):
    rows = scratch[:NBUF]
    gsem = scratch[NBUF:2 * NBUF]
    wsem = scratch[2 * NBUF:]
    wid = lax.axis_index("s") * 2 + lax.axis_index("c")
    base = wid * BPW
    pltpu.sync_copy(idx_hbm.at[wid], idx_v)

    def fire_gather(j, b):
        return pltpu.async_copy(table_hbm.at[idx_v.at[j]], rows[b], gsem[b])

    def fire_wb(j, b):
        return pltpu.async_copy(
            rows[b], out_hbm.at[pl.ds(base + j * CH, CH)], wsem[b])

    def wait_gather(j, b):
        pltpu.make_async_copy(table_hbm.at[idx_v.at[j]], rows[b], gsem[b]).wait()

    def wait_wb(j, b):
        pltpu.make_async_copy(
            rows[b], out_hbm.at[pl.ds(base + j * CH, CH)], wsem[b]).wait()

    def body(i, _):
        j0 = i * NBUF
        for b in range(NBUF):
            @pl.when(i > 0)
            def _(b=b):
                wait_wb(j0 - NBUF + b, b)
            fire_gather(j0 + b, b)
        for b in range(NBUF):
            wait_gather(j0 + b, b)
            fire_wb(j0 + b, b)
        return 0

    lax.fori_loop(0, NITER, body, 0)
    last = {b: (NITER - 1) * NBUF + b for b in range(NBUF)}
    for b in range(REMC):
        j = NITER * NBUF + b
        wait_wb(last[b], b)
        fire_gather(j, b)
        wait_gather(j, b)
        fire_wb(j, b)
        last[b] = j
    for b in range(NBUF):
        wait_wb(last[b], b)


@jax.jit
def _run(idx, table):
    mesh = plsc.VectorSubcoreMesh(core_axis_name="c", subcore_axis_name="s")
    f = functools.partial(
        pl.kernel,
        mesh=mesh,
        out_type=jax.ShapeDtypeStruct((B, DIM), jnp.float32),
        scratch_types=(
            [pltpu.VMEM((NCH, CH), jnp.int32)]
            + [pltpu.VMEM((CH, DIM), jnp.float32) for _ in range(NBUF)]
            + [pltpu.SemaphoreType.DMA for _ in range(2 * NBUF)]
        ),
    )(_gather_body)
    return f(idx, table)


def kernel(DNA, table):
    idx = DNA.reshape(NW, NCH, CH)
    out = _run(idx, table)
    return out.reshape(4, 8192, DIM)
